# baseline (speedup 1.0000x reference)
import jax
import jax.numpy as jnp
from jax.experimental import pallas as pl
from jax.experimental.pallas import tpu as pltpu

_B, _S, _D, _H = 128, 32, 128, 128
_C = 10.0
_NEG = -jnp.inf


def _decoder_kernel(f0_ref, lcvt_ref, nodet_ref, mask_ref,
                    gum_ref, liw_ref, Wc_ref, bc_ref, Wv_ref, bv_ref,
                    Wt_ref, bt_ref, Wq_ref, bq_ref, vp_ref,
                    logp_ref, idx_ref, last_ref, R_ref, init_ref,
                    T_ref, qh_ref, L_ref):
    nodext = nodet_ref[0]
    nodeyt = nodet_ref[1]
    lcvt = lcvt_ref[:]
    Wv = Wv_ref[:]
    Wq = Wq_ref[:]
    vp = vp_ref[:]
    bq = bq_ref[:]
    bv = bv_ref[:]

    lcv2 = lcvt.reshape(_S * _B, _D)
    T = (jnp.dot(lcv2, Wt_ref[:].T) + bt_ref[:]).reshape(_S, _B, _H)
    T_ref[:] = T
    h_bar = jnp.dot(jnp.mean(lcvt, axis=0), Wc_ref[:].T) + bc_ref[:]
    q0 = h_bar + (jnp.dot(liw_ref[:], Wv.T) + bv)

    row = jax.lax.broadcasted_iota(jnp.int32, (_S, _B), 0)

    def softmax_sample(lg, mask, g):
        lg = jnp.where(mask == 1.0, _NEG, lg)
        shifted = lg - jnp.max(lg, axis=0, keepdims=True)
        logp = shifted - jnp.log(jnp.sum(jnp.exp(shifted), axis=0, keepdims=True))
        score = lg + g
        smax = jnp.max(score, axis=0, keepdims=True)
        idx = jnp.min(jnp.where(score == smax, row, _S), axis=0)
        return idx, logp

    mask = mask_ref[:].T
    qh0 = jnp.dot(q0, Wq.T) + bq
    u0 = jnp.tanh(T + qh0[None, :, :])
    lg0 = _C * jnp.tanh(jnp.dot(u0.reshape(_S * _B, _H), vp).reshape(_S, _B))
    idx0, logp0 = softmax_sample(lg0, mask, gum_ref[0])
    idx0 = jnp.where(f0_ref[0] == 1, jnp.zeros_like(idx0), idx0)
    oh0 = row == idx0[None, :]
    slp0 = jnp.sum(jnp.where(oh0, logp0, 0.0), axis=0)
    mask = jnp.where(oh0, 1.0, mask)
    ohf0 = jnp.where(oh0, 1.0, 0.0)
    ih = jnp.sum(lcvt * ohf0[:, :, None], axis=0)
    nx0 = jnp.sum(jnp.where(oh0, nodext, 0.0), axis=0)
    ny0 = jnp.sum(jnp.where(oh0, nodeyt, 0.0), axis=0)
    cx, cy = nodext[0], nodeyt[0]
    init_ref[:] = jnp.concatenate([cx[:, None], cy[:, None]], axis=1)
    dx0, dy0 = nx0 - cx, ny0 - cy
    r0 = jnp.sqrt(dx0 * dx0 + dy0 * dy0)

    cat_all = jnp.concatenate(
        [jnp.broadcast_to(ih[None, :, :], (_S, _B, _D)), lcvt], axis=2)
    allq = h_bar[None, :, :] + (
        jnp.dot(cat_all.reshape(_S * _B, 2 * _D), Wv.T) + bv).reshape(_S, _B, _D)
    qh_ref[:] = (jnp.dot(allq.reshape(_S * _B, _D), Wq.T) + bq).reshape(_S, _B, _H)

    def build(p, _):
        qh = qh_ref[p]
        u = jnp.tanh(T_ref[:] + qh[None, :, :])
        lrow = _C * jnp.tanh(jnp.dot(u.reshape(_S * _B, _H), vp).reshape(_S, _B))
        L_ref[pl.ds(p, 1)] = lrow[None]
        return 0
    jax.lax.fori_loop(0, _S, build, 0, unroll=4)

    logp_acc = jnp.where(row == 0, slp0[None, :], 0.0)
    idx_acc = jnp.where(row == 0, idx0[None, :], 0)
    R_acc = jnp.where(row == 0, r0[None, :], 0.0)

    def body(i, carry):
        ohp, mask, cx, cy, logp_acc, idx_acc, R_acc = carry
        lg = jnp.sum(L_ref[:] * ohp[:, None, :], axis=0)
        idx, logp = softmax_sample(lg, mask, gum_ref[i])
        oh = row == idx[None, :]
        slp = jnp.sum(jnp.where(oh, logp, 0.0), axis=0)
        mask = jnp.where(oh, 1.0, mask)
        ohf = jnp.where(oh, 1.0, 0.0)
        nx = jnp.sum(jnp.where(oh, nodext, 0.0), axis=0)
        ny = jnp.sum(jnp.where(oh, nodeyt, 0.0), axis=0)
        dx, dy = nx - cx, ny - cy
        r = jnp.sqrt(dx * dx + dy * dy)
        sel = row == i
        logp_acc = jnp.where(sel, slp[None, :], logp_acc)
        idx_acc = jnp.where(sel, idx[None, :], idx_acc)
        R_acc = jnp.where(sel, r[None, :], R_acc)
        return ohf, mask, nx, ny, logp_acc, idx_acc, R_acc

    carry = (ohf0, mask, nx0, ny0, logp_acc, idx_acc, R_acc)
    _, mask, lx, ly, logp_acc, idx_acc, R_acc = jax.lax.fori_loop(
        1, _S, body, carry)

    logp_ref[:] = logp_acc.T
    idx_ref[:] = idx_acc.T
    R_ref[:] = R_acc.T
    last_ref[:] = jnp.concatenate([lx[:, None], ly[:, None]], axis=1)


def kernel(low_context_vector, original_node, mask, id, low_init_w, W_ctx,
           b_ctx, W_v, b_v, W_t, b_t, W_q, b_q, v_ptr):
    B, S, D, H = _B, _S, _D, _H
    with jax.ensure_compile_time_eval():
        skey = jax.random.key(42)
        gum = jnp.stack([
            jax.random.gumbel(jax.random.fold_in(skey, i), (B, S), jnp.float32).T
            for i in range(S)
        ])
    f0 = (jnp.asarray(id) == 0).astype(jnp.int32).reshape(1)
    lcvt = jnp.transpose(low_context_vector, (1, 0, 2))
    node_t = jnp.transpose(original_node, (2, 1, 0))

    out_shapes = (
        jax.ShapeDtypeStruct((B, S), jnp.float32),
        jax.ShapeDtypeStruct((B, S), jnp.int32),
        jax.ShapeDtypeStruct((B, 2), jnp.float32),
        jax.ShapeDtypeStruct((B, S), jnp.float32),
        jax.ShapeDtypeStruct((B, 2), jnp.float32),
    )
    vmem = pl.BlockSpec(memory_space=pltpu.VMEM)
    smem = pl.BlockSpec(memory_space=pltpu.SMEM)
    logp, idx, last, R, init = pl.pallas_call(
        _decoder_kernel,
        out_shape=out_shapes,
        in_specs=[smem] + [vmem] * 14,
        out_specs=(vmem, vmem, vmem, vmem, vmem),
        scratch_shapes=[
            pltpu.VMEM((S, B, H), jnp.float32),
            pltpu.VMEM((S, B, H), jnp.float32),
            pltpu.VMEM((S, S, B), jnp.float32),
        ],
    )(f0, lcvt, node_t, mask, gum,
      low_init_w.reshape(1, 2 * D), W_ctx, b_ctx, W_v, b_v,
      W_t, b_t, W_q, b_q, v_ptr.reshape(H, 1))

    return (logp, idx, init.reshape(B, 1, 2), last.reshape(B, 1, 2), R)


# speedup vs baseline: 1.0255x; 1.0255x over previous
import functools
import jax
import jax.numpy as jnp
from jax.experimental import pallas as pl
from jax.experimental.pallas import tpu as pltpu

_B, _S, _D, _H = 128, 32, 128, 128
_C = 10.0
_NEG = -jnp.inf
_dot = jnp.dot


def _decoder_kernel(f0_ref, lcvt_ref, nodet_ref, mask_ref,
                    gum_ref, liw_ref, Wc_ref, bc_ref, Wv_ref, bv_ref,
                    Wt_ref, bt_ref, Wq_ref, bq_ref, vp_ref,
                    logp_ref, idx_ref, last_ref, R_ref, init_ref,
                    T_ref, qh_ref, L_ref):
    nodext = nodet_ref[0]
    nodeyt = nodet_ref[1]
    lcvt = lcvt_ref[:]
    Wv = Wv_ref[:]
    Wq = Wq_ref[:]
    vp = vp_ref[:]
    bq = bq_ref[:]
    bv = bv_ref[:]

    lcv2 = lcvt.reshape(_S * _B, _D)
    T = (_dot(lcv2, Wt_ref[:].T) + bt_ref[:]).reshape(_S, _B, _H)
    T_ref[:] = T
    h_bar = _dot(jnp.mean(lcvt, axis=0), Wc_ref[:].T) + bc_ref[:]
    q0 = h_bar + (_dot(liw_ref[:], Wv.T) + bv)

    row = jax.lax.broadcasted_iota(jnp.int32, (_S, _B), 0)

    def softmax_sample(lg, mask, g):
        lg = jnp.where(mask == 1.0, _NEG, lg)
        shifted = lg - jnp.max(lg, axis=0, keepdims=True)
        logp = shifted - jnp.log(jnp.sum(jnp.exp(shifted), axis=0, keepdims=True))
        score = lg + g
        smax = jnp.max(score, axis=0, keepdims=True)
        idx = jnp.min(jnp.where(score == smax, row, _S), axis=0)
        return idx, logp

    mask = mask_ref[:].T
    qh0 = _dot(q0, Wq.T) + bq
    u0 = jnp.tanh(T + qh0[None, :, :])
    lg0 = _C * jnp.tanh(_dot(u0.reshape(_S * _B, _H), vp).reshape(_S, _B))
    idx0, logp0 = softmax_sample(lg0, mask, gum_ref[0])
    idx0 = jnp.where(f0_ref[0] == 1, jnp.zeros_like(idx0), idx0)
    oh0 = row == idx0[None, :]
    slp0 = jnp.sum(jnp.where(oh0, logp0, 0.0), axis=0)
    mask = jnp.where(oh0, 1.0, mask)
    ohf0 = jnp.where(oh0, 1.0, 0.0)
    ih = jnp.sum(lcvt * ohf0[:, :, None], axis=0)
    nx0 = jnp.sum(jnp.where(oh0, nodext, 0.0), axis=0)
    ny0 = jnp.sum(jnp.where(oh0, nodeyt, 0.0), axis=0)
    cx, cy = nodext[0], nodeyt[0]
    init_ref[:] = jnp.concatenate([cx[:, None], cy[:, None]], axis=1)
    dx0, dy0 = nx0 - cx, ny0 - cy
    r0 = jnp.sqrt(dx0 * dx0 + dy0 * dy0)

    cat_all = jnp.concatenate(
        [jnp.broadcast_to(ih[None, :, :], (_S, _B, _D)), lcvt], axis=2)
    allq = h_bar[None, :, :] + (
        _dot(cat_all.reshape(_S * _B, 2 * _D), Wv.T) + bv).reshape(_S, _B, _D)
    qh_ref[:] = (_dot(allq.reshape(_S * _B, _D), Wq.T) + bq).reshape(_S, _B, _H)

    def build(p, _):
        qh = qh_ref[p]
        u = jnp.tanh(T_ref[:] + qh[None, :, :])
        lrow = _C * jnp.tanh(_dot(u.reshape(_S * _B, _H), vp).reshape(_S, _B))
        L_ref[pl.ds(p, 1)] = lrow[None]
        return 0
    jax.lax.fori_loop(0, _S, build, 0, unroll=8)

    logp_acc = jnp.where(row == 0, slp0[None, :], 0.0)
    idx_acc = jnp.where(row == 0, idx0[None, :], 0)
    R_acc = jnp.where(row == 0, r0[None, :], 0.0)

    def body(i, carry):
        ohp, mask, cx, cy, logp_acc, idx_acc, R_acc = carry
        lg = jnp.sum(L_ref[:] * ohp[:, None, :], axis=0)
        idx, logp = softmax_sample(lg, mask, gum_ref[i])
        oh = row == idx[None, :]
        slp = jnp.sum(jnp.where(oh, logp, 0.0), axis=0)
        mask = jnp.where(oh, 1.0, mask)
        ohf = jnp.where(oh, 1.0, 0.0)
        nx = jnp.sum(jnp.where(oh, nodext, 0.0), axis=0)
        ny = jnp.sum(jnp.where(oh, nodeyt, 0.0), axis=0)
        dx, dy = nx - cx, ny - cy
        r = jnp.sqrt(dx * dx + dy * dy)
        sel = row == i
        logp_acc = jnp.where(sel, slp[None, :], logp_acc)
        idx_acc = jnp.where(sel, idx[None, :], idx_acc)
        R_acc = jnp.where(sel, r[None, :], R_acc)
        return ohf, mask, nx, ny, logp_acc, idx_acc, R_acc

    carry = (ohf0, mask, nx0, ny0, logp_acc, idx_acc, R_acc)
    _, mask, lx, ly, logp_acc, idx_acc, R_acc = jax.lax.fori_loop(
        1, _S, body, carry)

    logp_ref[:] = logp_acc.T
    idx_ref[:] = idx_acc.T
    R_ref[:] = R_acc.T
    last_ref[:] = jnp.concatenate([lx[:, None], ly[:, None]], axis=1)


def kernel(low_context_vector, original_node, mask, id, low_init_w, W_ctx,
           b_ctx, W_v, b_v, W_t, b_t, W_q, b_q, v_ptr):
    B, S, D, H = _B, _S, _D, _H
    with jax.ensure_compile_time_eval():
        skey = jax.random.key(42)
        gum = jnp.stack([
            jax.random.gumbel(jax.random.fold_in(skey, i), (B, S), jnp.float32).T
            for i in range(S)
        ])
    f0 = (jnp.asarray(id) == 0).astype(jnp.int32).reshape(1)
    lcvt = jnp.transpose(low_context_vector, (1, 0, 2))
    node_t = jnp.transpose(original_node, (2, 1, 0))

    out_shapes = (
        jax.ShapeDtypeStruct((B, S), jnp.float32),
        jax.ShapeDtypeStruct((B, S), jnp.int32),
        jax.ShapeDtypeStruct((B, 2), jnp.float32),
        jax.ShapeDtypeStruct((B, S), jnp.float32),
        jax.ShapeDtypeStruct((B, 2), jnp.float32),
    )
    vmem = pl.BlockSpec(memory_space=pltpu.VMEM)
    smem = pl.BlockSpec(memory_space=pltpu.SMEM)
    logp, idx, last, R, init = pl.pallas_call(
        _decoder_kernel,
        out_shape=out_shapes,
        in_specs=[smem] + [vmem] * 14,
        out_specs=(vmem, vmem, vmem, vmem, vmem),
        scratch_shapes=[
            pltpu.VMEM((S, B, H), jnp.float32),
            pltpu.VMEM((S, B, H), jnp.float32),
            pltpu.VMEM((S, S, B), jnp.float32),
        ],
    )(f0, lcvt, node_t, mask, gum,
      low_init_w.reshape(1, 2 * D), W_ctx, b_ctx, W_v, b_v,
      W_t, b_t, W_q, b_q, v_ptr.reshape(H, 1))

    return (logp, idx, init.reshape(B, 1, 2), last.reshape(B, 1, 2), R)
